# baseline (speedup 1.0000x reference)
import functools

import jax
import jax.numpy as jnp
from jax.experimental import pallas as pl
from jax.experimental.pallas import tpu as pltpu

_MAX_POSITION = 512
_NLAG = 8


def _build_kernel(table_ref, e2_ref, *, hidden, p, base, ncopy):
    e2_rows = e2_ref.shape[0]
    e2_ref[...] = jnp.broadcast_to(
        table_ref[2 * p : 2 * p + 1, :], (e2_rows, hidden)
    )
    e2_ref[0:ncopy, :] = table_ref[base : base + ncopy, :]


def _copy_kernel(e2_hbm, out_ref, e2_vmem, load_sem, sems,
                 *, len_q, len_k, hidden, p, base):
    load = pltpu.make_async_copy(e2_hbm, e2_vmem, load_sem)
    load.start()
    load.wait()

    row = len_k * hidden

    def copy_for(idx):
        shift = (p - base) - idx
        return pltpu.make_async_copy(
            e2_vmem.at[pl.ds(shift * hidden, row)],
            out_ref.at[pl.ds(idx * row, row)],
            sems.at[idx % _NLAG],
        )

    for idx in range(len_q):
        if idx >= _NLAG:
            copy_for(idx - _NLAG).wait()
        copy_for(idx).start()
    for idx in range(max(len_q - _NLAG, 0), len_q):
        copy_for(idx).wait()


def kernel(q, k, embeddings_table):
    len_q = q.shape[1]
    len_k = k.shape[1]
    hidden = embeddings_table.shape[1]
    p = _MAX_POSITION
    base = ((p - len_q) // 8) * 8
    ncopy = ((2 * p - base) // 8) * 8
    max_shift = p - base
    e2_rows = ((max_shift + len_k + 7) // 8) * 8

    build = functools.partial(
        _build_kernel, hidden=hidden, p=p, base=base, ncopy=ncopy
    )
    e2 = pl.pallas_call(
        build,
        in_specs=[pl.BlockSpec(embeddings_table.shape, lambda: (0, 0))],
        out_specs=pl.BlockSpec((e2_rows, hidden), lambda: (0, 0)),
        out_shape=jax.ShapeDtypeStruct((e2_rows, hidden), jnp.float32),
    )(embeddings_table)

    copy = functools.partial(
        _copy_kernel, len_q=len_q, len_k=len_k, hidden=hidden, p=p, base=base
    )
    flat = pl.pallas_call(
        copy,
        in_specs=[pl.BlockSpec(memory_space=pl.ANY)],
        out_specs=pl.BlockSpec(memory_space=pl.ANY),
        out_shape=jax.ShapeDtypeStruct((len_q * len_k * hidden,), jnp.float32),
        scratch_shapes=[
            pltpu.VMEM((e2_rows * hidden,), jnp.float32),
            pltpu.SemaphoreType.DMA,
            pltpu.SemaphoreType.DMA((_NLAG,)),
        ],
    )(e2.reshape(-1))
    return flat.reshape(len_q, len_k, hidden)


# speedup vs baseline: 1.7030x; 1.7030x over previous
import functools

import jax
import jax.numpy as jnp
from jax.experimental import pallas as pl
from jax.experimental.pallas import tpu as pltpu

_MAX_POSITION = 512
_CHUNK = 256


def _build_kernel(table_ref, e2_ref, *, hidden, p, base, ncopy):
    e2_rows = e2_ref.shape[0]
    e2_ref[...] = jnp.broadcast_to(
        table_ref[2 * p : 2 * p + 1, :], (e2_rows, hidden)
    )
    e2_ref[0:ncopy, :] = table_ref[base : base + ncopy, :]


def _copy_kernel(e2_ref, out_ref, *, len_q, p, base):
    i = pl.program_id(0)
    c = pl.program_id(1)
    shift = (p - base) - i
    s8 = pl.multiple_of((shift // 8) * 8, 8)
    r = shift % 8
    win = _CHUNK + 8
    a = e2_ref[pl.ds(s8 + c * _CHUNK, win), :]
    rolled = pltpu.roll(a, (-r) % win, axis=0)
    out_ref[0, :, :] = rolled[0:_CHUNK, :]


def kernel(q, k, embeddings_table):
    len_q = q.shape[1]
    len_k = k.shape[1]
    hidden = embeddings_table.shape[1]
    p = _MAX_POSITION
    base = ((p - len_q) // 8) * 8
    ncopy = ((2 * p - base) // 8) * 8
    max_shift = p - base
    e2_rows = ((max_shift + len_k + 8 + 7) // 8) * 8
    n_chunks = len_k // _CHUNK

    build = functools.partial(
        _build_kernel, hidden=hidden, p=p, base=base, ncopy=ncopy
    )
    e2 = pl.pallas_call(
        build,
        in_specs=[pl.BlockSpec(embeddings_table.shape, lambda: (0, 0))],
        out_specs=pl.BlockSpec((e2_rows, hidden), lambda: (0, 0)),
        out_shape=jax.ShapeDtypeStruct((e2_rows, hidden), jnp.float32),
    )(embeddings_table)

    copy = functools.partial(_copy_kernel, len_q=len_q, p=p, base=base)
    return pl.pallas_call(
        copy,
        grid=(len_q, n_chunks),
        in_specs=[pl.BlockSpec((e2_rows, hidden), lambda i, c: (0, 0))],
        out_specs=pl.BlockSpec((1, _CHUNK, hidden), lambda i, c: (i, c, 0)),
        out_shape=jax.ShapeDtypeStruct((len_q, len_k, hidden), jnp.float32),
        compiler_params=pltpu.CompilerParams(
            dimension_semantics=("parallel", "parallel"),
        ),
    )(e2)
